# baseline (speedup 1.0000x reference)
import jax
import jax.numpy as jnp
from jax import lax
from jax.experimental import pallas as pl
from jax.experimental.pallas import tpu as pltpu
from jax.experimental.pallas import tpu_sc as plsc

_NS = 601
_EPS = 0.001
_NC = 2
_NSUB = 16
_NW = _NC * _NSUB
_CH = 64
_NROWS = 4096 * 50
_RPW = _NROWS // _NW
_NCHUNK = _RPW // _CH
_CHW = _CH * _NS


def _c_kernel(x_ref, o_ref):
    x = x_ref[...]
    t = jnp.sign(x) * (jnp.sqrt(jnp.abs(x) + 1.0) - 1.0 + _EPS * x)
    o_ref[...] = jnp.clip(t + 300.0, 0.0, 599.999)


def _sc_body(c_hbm, out_hbm, cbuf, rowbuf):
    cid = lax.axis_index("c")
    sid = lax.axis_index("s")
    w = sid * _NC + cid
    base = w * _RPW

    def _zero(i, carry):
        rowbuf[pl.ds(i * 16, 16)] = jnp.zeros((16,), jnp.float32)
        return carry

    lax.fori_loop(0, _CHW // 16, _zero, 0)

    def _chunk(g, carry):
        row0 = base + g * _CH
        pltpu.sync_copy(c_hbm.at[pl.ds(row0, _CH)], cbuf)
        for sub in range(_CH // 16):
            c = cbuf[pl.ds(sub * 16, 16)]
            li = jnp.minimum(c.astype(jnp.int32), _NS - 2)
            p_low = (li + 1).astype(jnp.float32) - c
            p_high = 1.0 - p_low
            idx = (lax.iota(jnp.int32, 16) + sub * 16) * _NS + li
            plsc.store_scatter(rowbuf, [idx], p_low)
            plsc.store_scatter(rowbuf, [idx + 1], p_high)
        pltpu.sync_copy(rowbuf, out_hbm.at[pl.ds(row0 * _NS, _CHW)])
        z = jnp.zeros((16,), jnp.float32)
        for sub in range(_CH // 16):
            c = cbuf[pl.ds(sub * 16, 16)]
            li = jnp.minimum(c.astype(jnp.int32), _NS - 2)
            idx = (lax.iota(jnp.int32, 16) + sub * 16) * _NS + li
            plsc.store_scatter(rowbuf, [idx], z)
            plsc.store_scatter(rowbuf, [idx + 1], z)
        return carry

    lax.fori_loop(0, _NCHUNK, _chunk, 0)


@jax.jit
def kernel(target_value):
    b, k = target_value.shape
    c = pl.pallas_call(
        _c_kernel,
        out_shape=jax.ShapeDtypeStruct((b, k), jnp.float32),
    )(target_value)
    sc_call = pl.kernel(
        _sc_body,
        out_type=jax.ShapeDtypeStruct((_NROWS * _NS,), jnp.float32),
        mesh=plsc.VectorSubcoreMesh(core_axis_name="c", subcore_axis_name="s"),
        compiler_params=pltpu.CompilerParams(needs_layout_passes=False),
        scratch_types=[
            pltpu.VMEM((_CH,), jnp.float32),
            pltpu.VMEM((_CHW,), jnp.float32),
        ],
    )
    out_flat = sc_call(c.reshape(b * k))
    return out_flat.reshape(b, k, _NS)


# speedup vs baseline: 15.9532x; 15.9532x over previous
import jax
import jax.numpy as jnp
from jax.experimental import pallas as pl

_SUPPORTS_MIN = -300.0
_NUM_SUPPORTS = 601
_EPSILON = 0.001


def _two_hot_kernel(x_ref, o_ref):
    x = x_ref[...]
    t = jnp.sign(x) * (jnp.sqrt(jnp.abs(x) + 1.0) - 1.0 + _EPSILON * x)
    c = jnp.clip(t - _SUPPORTS_MIN, 0.0, float(_NUM_SUPPORTS - 1))
    jf = jax.lax.broadcasted_iota(
        jnp.int32, (_NUM_SUPPORTS, x.shape[1]), 0).astype(jnp.float32)
    o_ref[...] = jnp.maximum(0.0, 1.0 - jnp.abs(jf - c))


@jax.jit
def kernel(target_value):
    b, k = target_value.shape
    xt = target_value.T.reshape(k, 1, b)
    out_p = pl.pallas_call(
        _two_hot_kernel,
        grid=(k,),
        in_specs=[pl.BlockSpec((None, 1, b), lambda i: (i, 0, 0))],
        out_specs=pl.BlockSpec((None, _NUM_SUPPORTS, b), lambda i: (i, 0, 0)),
        out_shape=jax.ShapeDtypeStruct((k, _NUM_SUPPORTS, b), jnp.float32),
    )(xt)
    return out_p.transpose(2, 0, 1)
